# baseline (speedup 1.0000x reference)
import functools

import jax
import jax.numpy as jnp
from jax import lax
from jax.experimental import pallas as pl
from jax.experimental.pallas import tpu as pltpu
from jax.experimental.pallas import tpu_sc as plsc

VOCAB = 1000000
DIM = 32
TOTAL = 204800
W = 128
ROWS = TOTAL // W
NC, NS = 2, 16
NW = NC * NS
RPW = ROWS // NW
CH = 10
NCHUNK = RPW // CH


def _body(idx_hbm, table_hbm, out_hbm, idx_v, rows_v, sem):
    wid = lax.axis_index("s") * NC + lax.axis_index("c")
    pltpu.sync_copy(idx_hbm.at[wid], idx_v)

    def chunk(c, carry):
        row0 = c * CH
        copies = []
        for j in range(CH):
            copies.append(
                pltpu.async_copy(table_hbm.at[idx_v.at[row0 + j]],
                                 rows_v.at[j], sem))
        for cp in copies:
            cp.wait()
        pltpu.sync_copy(rows_v, out_hbm.at[wid, pl.ds(row0, CH)])
        return carry

    lax.fori_loop(0, NCHUNK, chunk, 0)


@functools.partial(jax.jit, static_argnames=())
def _gather(idx2d, table):
    k = pl.kernel(
        _body,
        out_type=jax.ShapeDtypeStruct((NW, RPW, W, DIM), jnp.float32),
        mesh=plsc.VectorSubcoreMesh(core_axis_name="c", subcore_axis_name="s"),
        scratch_types=[
            pltpu.VMEM((RPW, W), jnp.int32),
            pltpu.VMEM((CH, W, DIM), jnp.float32),
            pltpu.SemaphoreType.DMA,
        ],
        compiler_params=pltpu.CompilerParams(use_tc_tiling_on_sc=False),
    )
    return k(idx2d, table)


def kernel(indices, row_splits, table):
    del row_splits
    idx2d = indices.reshape(NW, RPW, W)
    out = _gather(idx2d, table)
    return out.reshape(TOTAL, DIM)
